# baseline (speedup 1.0000x reference)
import jax
import jax.numpy as jnp
from jax.experimental import pallas as pl
from jax.experimental.pallas import tpu as pltpu

_TOTAL_ROWS = 2 * 4096
_SRC_OFFSET = 1 * 4096
_BUF_ROWS = 1024
_R = 3
_D = 1

_DIRECT_ROWS = 128
_RING_ROWS = _TOTAL_ROWS - _DIRECT_ROWS

_CHUNKS = [128, 256, 576] + [1024] * 6 + [576, 256, 128]
_OFFS = [0]
for _c in _CHUNKS:
    _OFFS.append(_OFFS[-1] + _c)
assert _OFFS[-1] == _RING_ROWS
_N = len(_CHUNKS)


def _copy_body(x_hbm, out_hbm, *scratch):
    bufs = scratch[:_R]
    in_sems = scratch[_R]
    out_sems = scratch[_R + 1]
    direct_sem = scratch[_R + 2]

    direct = pltpu.make_async_copy(
        x_hbm.at[pl.ds(_SRC_OFFSET + _RING_ROWS, _DIRECT_ROWS)],
        out_hbm.at[pl.ds(_RING_ROWS, _DIRECT_ROWS)],
        direct_sem,
    )
    direct.start()

    def mk_in(i):
        return pltpu.make_async_copy(
            x_hbm.at[pl.ds(_SRC_OFFSET + _OFFS[i], _CHUNKS[i])],
            bufs[i % _R].at[pl.ds(0, _CHUNKS[i])],
            in_sems.at[i % _R],
        )

    def mk_out(i):
        return pltpu.make_async_copy(
            bufs[i % _R].at[pl.ds(0, _CHUNKS[i])],
            out_hbm.at[pl.ds(_OFFS[i], _CHUNKS[i])],
            out_sems.at[i % _R],
        )

    for i in range(_R):
        mk_in(i).start()

    waited_out = set()
    for i in range(_N):
        mk_in(i).wait()
        mk_out(i).start()
        j = i - _D
        if j >= 0 and j + _R < _N:
            mk_out(j).wait()
            waited_out.add(j)
            mk_in(j + _R).start()
    for i in range(_N):
        if i not in waited_out:
            mk_out(i).wait()
    direct.wait()


def kernel(x):
    x2 = x.reshape(4 * 4096, 4096)
    out = pl.pallas_call(
        _copy_body,
        in_specs=[pl.BlockSpec(memory_space=pl.ANY)],
        out_specs=pl.BlockSpec(memory_space=pl.ANY),
        out_shape=jax.ShapeDtypeStruct((_TOTAL_ROWS, 4096), jnp.float32),
        scratch_shapes=(
            [pltpu.VMEM((_BUF_ROWS, 4096), jnp.float32) for _ in range(_R)]
            + [pltpu.SemaphoreType.DMA((_R,)), pltpu.SemaphoreType.DMA((_R,)),
               pltpu.SemaphoreType.DMA]
        ),
    )(x2)
    return out.reshape(2, 4096, 4096)


# speedup vs baseline: 1.0078x; 1.0078x over previous
import jax
import jax.numpy as jnp
from jax.experimental import pallas as pl
from jax.experimental.pallas import tpu as pltpu

_TOTAL_ROWS = 2 * 4096
_SRC_OFFSET = 1 * 4096
_BUF_ROWS = 768
_R = 4
_D = 1

_CHUNKS = [128, 512] + [768] * 9 + [512, 128]
_OFFS = [0]
for _c in _CHUNKS:
    _OFFS.append(_OFFS[-1] + _c)
assert _OFFS[-1] == _TOTAL_ROWS
_N = len(_CHUNKS)


def _copy_body(x_hbm, out_hbm, *scratch):
    bufs = scratch[:_R]
    in_sems = scratch[_R]
    out_sems = scratch[_R + 1]

    def mk_in(i):
        return pltpu.make_async_copy(
            x_hbm.at[pl.ds(_SRC_OFFSET + _OFFS[i], _CHUNKS[i])],
            bufs[i % _R].at[pl.ds(0, _CHUNKS[i])],
            in_sems.at[i % _R],
        )

    def mk_out(i):
        return pltpu.make_async_copy(
            bufs[i % _R].at[pl.ds(0, _CHUNKS[i])],
            out_hbm.at[pl.ds(_OFFS[i], _CHUNKS[i])],
            out_sems.at[i % _R],
        )

    for i in range(_R):
        mk_in(i).start()

    waited_out = set()
    for i in range(_N):
        mk_in(i).wait()
        mk_out(i).start()
        j = i - _D
        if j >= 0 and j + _R < _N:
            mk_out(j).wait()
            waited_out.add(j)
            mk_in(j + _R).start()
    for i in range(_N):
        if i not in waited_out:
            mk_out(i).wait()


def kernel(x):
    x2 = x.reshape(4 * 4096, 4096)
    out = pl.pallas_call(
        _copy_body,
        in_specs=[pl.BlockSpec(memory_space=pl.ANY)],
        out_specs=pl.BlockSpec(memory_space=pl.ANY),
        out_shape=jax.ShapeDtypeStruct((_TOTAL_ROWS, 4096), jnp.float32),
        scratch_shapes=(
            [pltpu.VMEM((_BUF_ROWS, 4096), jnp.float32) for _ in range(_R)]
            + [pltpu.SemaphoreType.DMA((_R,)), pltpu.SemaphoreType.DMA((_R,))]
        ),
    )(x2)
    return out.reshape(2, 4096, 4096)
